# Initial kernel scaffold; baseline (speedup 1.0000x reference)
import jax
import jax.numpy as jnp
from jax.experimental import pallas as pl


def kernel(x, edge_index, W1, b1, W2, b2, W3, b3, W4, b4):
    raise NotImplementedError("write your pallas kernel here")


# speedup vs baseline: 6.9904x; 6.9904x over previous
import functools

import jax
import jax.numpy as jnp
from jax import lax
from jax.experimental import pallas as pl
from jax.experimental.pallas import tpu as pltpu
from jax.experimental.pallas import tpu_sc as plsc

N_NODES = 10000
N_PAD = 10240
NC = 2
NS = 16
NW = NC * NS
K = 128
ZR = 128
ROWS_PER_TILE = N_PAD // NS
ZCHUNKS = ROWS_PER_TILE // ZR


def _agg_body(D, with_counts, n_chunks, *refs):
    if with_counts:
        (x_hbm, src_hbm, dst_hbm, out_hbm, cnt_hbm,
         srcb, dstb, rows, zbuf, onesb, zcnt, acc, cacc, sem) = refs
    else:
        (x_hbm, src_hbm, dst_hbm, out_hbm,
         srcb, dstb, rows, zbuf, acc, sem) = refs

    c = lax.axis_index("c")
    s = lax.axis_index("s")
    w = c * NS + s

    def zrow(i, carry):
        for jj in range(D // 16):
            zbuf[i, pl.ds(jj * 16, 16)] = jnp.zeros((16,), jnp.float32)
        return carry
    lax.fori_loop(0, ZR, zrow, 0)
    for b in range(ZCHUNKS):
        pltpu.sync_copy(zbuf, acc.at[pl.ds((s * ZCHUNKS + b) * ZR, ZR)])

    if with_counts:
        for jj in range(K // 16):
            onesb[pl.ds(jj * 16, 16)] = jnp.ones((16,), jnp.float32)
        for jj in range(ROWS_PER_TILE // 16):
            zcnt[pl.ds(jj * 16, 16)] = jnp.zeros((16,), jnp.float32)
        pltpu.sync_copy(zcnt, cacc.at[pl.ds(s * ROWS_PER_TILE, ROWS_PER_TILE)])

    plsc.subcore_barrier()

    base_chunks = n_chunks // NW
    rem_chunks = n_chunks % NW

    def do_chunk(cj):
        off = pl.multiple_of(cj * K, K)
        pltpu.sync_copy(src_hbm.at[pl.ds(off, K)], srcb)
        pltpu.sync_copy(dst_hbm.at[pl.ds(off, K)], dstb)
        pltpu.async_copy(x_hbm.at[srcb], rows, sem).wait()
        pltpu.sync_copy(rows, acc.at[dstb], add=True)
        if with_counts:
            pltpu.sync_copy(onesb, cacc.at[dstb], add=True)

    def body(j, carry):
        do_chunk(w + NW * j)
        return carry
    lax.fori_loop(0, base_chunks, body, 0)
    if rem_chunks:
        @pl.when(w < rem_chunks)
        def _():
            do_chunk(w + NW * base_chunks)

    plsc.subcore_barrier()

    for b in range(ZCHUNKS):
        r0 = (s * ZCHUNKS + b) * ZR
        pltpu.sync_copy(acc.at[pl.ds(r0, ZR)],
                        out_hbm.at[pl.ds(c * N_PAD + r0, ZR)])
    if with_counts:
        r0 = s * ROWS_PER_TILE
        pltpu.sync_copy(cacc.at[pl.ds(r0, ROWS_PER_TILE)],
                        cnt_hbm.at[pl.ds(c * N_PAD + r0, ROWS_PER_TILE)])


def _sc_aggregate(x, src, dst, with_counts=False):
    e = src.shape[0]
    assert e % K == 0
    n_chunks = e // K
    D = x.shape[1]
    mesh = plsc.VectorSubcoreMesh(core_axis_name="c", subcore_axis_name="s")

    out_type = [jax.ShapeDtypeStruct((NC * N_PAD, D), jnp.float32)]
    scratch = [
        pltpu.VMEM((K,), jnp.int32),
        pltpu.VMEM((K,), jnp.int32),
        pltpu.VMEM((K, D), jnp.float32),
        pltpu.VMEM((ZR, D), jnp.float32),
    ]
    if with_counts:
        out_type.append(jax.ShapeDtypeStruct((NC * N_PAD,), jnp.float32))
        scratch += [
            pltpu.VMEM((K,), jnp.float32),
            pltpu.VMEM((ROWS_PER_TILE,), jnp.float32),
        ]
    scratch.append(pltpu.VMEM_SHARED((N_PAD, D), jnp.float32))
    if with_counts:
        scratch.append(pltpu.VMEM_SHARED((N_PAD,), jnp.float32))
    scratch.append(pltpu.SemaphoreType.DMA)

    body = functools.partial(_agg_body, D, with_counts, n_chunks)
    params = pltpu.CompilerParams(use_tc_tiling_on_sc=(D % 128 == 0))
    fn = pl.kernel(body, mesh=mesh, out_type=tuple(out_type),
                   scratch_types=scratch, compiler_params=params)
    res = fn(x, src, dst)
    if with_counts:
        p, cnt = res
        return p.reshape(NC, N_PAD, D), cnt.reshape(NC, N_PAD)
    return res[0].reshape(NC, N_PAD, D) if isinstance(res, (tuple, list)) else res.reshape(NC, N_PAD, D)


R = 1024
GRID = N_PAD // R


def _mean(p, cnt):
    summed = p[0] + p[1]
    c = jnp.maximum(cnt[0] + cnt[1], 1.0)
    return summed / c[:, None]


def _matT(a, w):
    return lax.dot_general(a, w, (((1,), (1,)), ((), ())),
                           preferred_element_type=jnp.float32)


def _l2relu(o):
    nrm = jnp.sqrt(jnp.sum(o * o, axis=1, keepdims=True))
    return jnp.maximum(o / jnp.maximum(nrm, 1e-12), 0.0)


def _dense1_body(p_ref, cnt_ref, w1_ref, b1_ref, w2_ref, out_ref):
    mean = _mean(p_ref[...], cnt_ref[...])
    h = _l2relu(_matT(mean, w1_ref[...]) + b1_ref[...])
    out_ref[...] = _matT(h, w2_ref[...])


def _dense2_body(p_ref, cnt_ref, b2_ref, out_ref):
    mean = _mean(p_ref[...], cnt_ref[...])
    out_ref[...] = _l2relu(mean + b2_ref[...])


def _dense3_body(p_ref, cnt_ref, w3_ref, b3_ref, out_ref):
    mean = _mean(p_ref[...], cnt_ref[...])
    out_ref[...] = _l2relu(_matT(mean, w3_ref[...]) + b3_ref[...])


def _dense4_body(p_ref, cnt_ref, w4_ref, b4_ref, out_ref):
    mean = _mean(p_ref[...], cnt_ref[...])
    o = _matT(mean, w4_ref[...]) + b4_ref[...]
    m = jnp.max(o, axis=1, keepdims=True)
    e = jnp.exp(o - m)
    out_ref[...] = e / jnp.sum(e, axis=1, keepdims=True)


def _part_spec(D):
    return pl.BlockSpec((NC, R, D), lambda i: (0, i, 0))


def _cnt_spec():
    return pl.BlockSpec((NC, R), lambda i: (0, i))


def _full_spec(shape):
    nd = len(shape)
    return pl.BlockSpec(shape, lambda i: (0,) * nd)


def _run_dense(body, p, cnt, weights, d_out):
    d_in = p.shape[-1]
    in_specs = [_part_spec(d_in), _cnt_spec()] + [_full_spec(w.shape) for w in weights]
    return pl.pallas_call(
        body,
        grid=(GRID,),
        in_specs=in_specs,
        out_specs=pl.BlockSpec((R, d_out), lambda i: (i, 0)),
        out_shape=jax.ShapeDtypeStruct((N_PAD, d_out), jnp.float32),
    )(p, cnt, *weights)


def kernel(x, edge_index, W1, b1, W2, b2, W3, b3, W4, b4):
    src = edge_index[0]
    dst = edge_index[1]
    b1r = b1.reshape(1, -1)
    b2r = b2.reshape(1, -1)
    b3r = b3.reshape(1, -1)
    b4r = b4.reshape(1, -1)

    p0, cnt = _sc_aggregate(x, src, dst, with_counts=True)
    p2in = _run_dense(_dense1_body, p0, cnt, [W1, b1r, W2], 64)
    q2 = _sc_aggregate(p2in, src, dst)
    h2 = _run_dense(_dense2_body, q2, cnt, [b2r], 64)
    q3 = _sc_aggregate(h2, src, dst)
    h3 = _run_dense(_dense3_body, q3, cnt, [W3, b3r], 128)
    q4 = _sc_aggregate(h3, src, dst)
    out = _run_dense(_dense4_body, q4, cnt, [W4, b4r], 128)
    return out[:N_NODES]


# speedup vs baseline: 11.1114x; 1.5895x over previous
import functools

import jax
import jax.numpy as jnp
from jax import lax
from jax.experimental import pallas as pl
from jax.experimental.pallas import tpu as pltpu
from jax.experimental.pallas import tpu_sc as plsc

N_NODES = 10000
N_PAD = 10240
NC = 2
NS = 16
K = 128
NB = 5
ROWS_PER_TILE = N_PAD // NS
ZCH = ROWS_PER_TILE // K


def _agg_body(Dh, with_counts, cpw, *refs):
    if with_counts:
        (x0_hbm, x1_hbm, src_hbm, dst_hbm, out_hbm, cnt_hbm, *rest) = refs
    else:
        (x0_hbm, x1_hbm, src_hbm, dst_hbm, out_hbm, *rest) = refs
    srcb, dstb = rest[0], rest[1]
    rows = rest[2:2 + NB]
    i = 2 + NB
    if with_counts:
        onesb, zcnt = rest[i], rest[i + 1]
        i += 2
    acc = rest[i]
    i += 1
    if with_counts:
        cacc = rest[i]
        i += 1
    gsem = rest[i:i + NB]
    ssem = rest[i + NB:i + 2 * NB]
    csem = rest[i + 2 * NB:i + 3 * NB]

    c = lax.axis_index("c")
    s = lax.axis_index("s")

    def zrow(r, carry):
        for jj in range(Dh // 16):
            rows[0][r, pl.ds(jj * 16, 16)] = jnp.zeros((16,), jnp.float32)
        return carry
    lax.fori_loop(0, K, zrow, 0)
    for b in range(ZCH):
        pltpu.sync_copy(rows[0], acc.at[pl.ds((s * ZCH + b) * K, K)])

    if with_counts:
        for jj in range(K // 16):
            onesb[pl.ds(jj * 16, 16)] = jnp.ones((16,), jnp.float32)
        for jj in range(ROWS_PER_TILE // 16):
            zcnt[pl.ds(jj * 16, 16)] = jnp.zeros((16,), jnp.float32)
        pltpu.sync_copy(zcnt, cacc.at[pl.ds(s * ROWS_PER_TILE, ROWS_PER_TILE)])

    base = s * cpw
    pltpu.sync_copy(src_hbm.at[pl.ds(base, cpw)], srcb)
    pltpu.sync_copy(dst_hbm.at[pl.ds(base, cpw)], dstb)

    plsc.subcore_barrier()

    def run_loop(table):
        def group(g, carry):
            j0 = g * NB
            gh = [pltpu.async_copy(table.at[srcb.at[j0 + b]], rows[b], gsem[b])
                  for b in range(NB)]
            sh = []
            for b in range(NB):
                gh[b].wait()
                sh.append(pltpu.async_copy(rows[b], acc.at[dstb.at[j0 + b]],
                                           ssem[b], add=True))
                if with_counts:
                    sh.append(pltpu.async_copy(onesb, cacc.at[dstb.at[j0 + b]],
                                               csem[b], add=True))
            for h in sh:
                h.wait()
            return carry
        lax.fori_loop(0, cpw // NB, group, 0)

    @pl.when(c == 0)
    def _():
        run_loop(x0_hbm)

    @pl.when(c == 1)
    def _():
        run_loop(x1_hbm)

    plsc.subcore_barrier()

    for b in range(ZCH):
        r0 = (s * ZCH + b) * K
        pltpu.sync_copy(acc.at[pl.ds(r0, K)],
                        out_hbm.at[pl.ds(c * N_PAD + r0, K)])
    if with_counts:
        r0 = s * ROWS_PER_TILE
        pltpu.sync_copy(cacc.at[pl.ds(r0, ROWS_PER_TILE)],
                        cnt_hbm.at[pl.ds(c * N_PAD + r0, ROWS_PER_TILE)])


def _sc_aggregate(x01, src2, dst2, with_counts=False):
    n_chunks = src2.shape[0]
    cpw = n_chunks // NS
    assert n_chunks == cpw * NS and cpw % NB == 0
    Dh = x01[0].shape[1]
    mesh = plsc.VectorSubcoreMesh(core_axis_name="c", subcore_axis_name="s")

    out_type = [jax.ShapeDtypeStruct((NC * N_PAD, Dh), jnp.float32)]
    scratch = [
        pltpu.VMEM((cpw, K), jnp.int32),
        pltpu.VMEM((cpw, K), jnp.int32),
    ]
    scratch += [pltpu.VMEM((K, Dh), jnp.float32) for _ in range(NB)]
    if with_counts:
        out_type.append(jax.ShapeDtypeStruct((NC * N_PAD,), jnp.float32))
        scratch += [
            pltpu.VMEM((K,), jnp.float32),
            pltpu.VMEM((ROWS_PER_TILE,), jnp.float32),
        ]
    scratch.append(pltpu.VMEM_SHARED((N_PAD, Dh), jnp.float32))
    if with_counts:
        scratch.append(pltpu.VMEM_SHARED((N_PAD,), jnp.float32))
    nsem = 3 * NB if with_counts else 2 * NB
    scratch += [pltpu.SemaphoreType.DMA] * nsem

    body = functools.partial(_agg_body, Dh, with_counts, cpw)
    params = pltpu.CompilerParams(use_tc_tiling_on_sc=False)
    fn = pl.kernel(body, mesh=mesh, out_type=tuple(out_type),
                   scratch_types=scratch, compiler_params=params)
    res = fn(x01[0], x01[1], src2, dst2)
    if with_counts:
        p, cnt = res
        return p.reshape(NC, N_PAD, Dh), cnt.reshape(NC, N_PAD)
    res = res[0] if isinstance(res, (tuple, list)) else res
    return res.reshape(NC, N_PAD, Dh)


R = 1024
GRID = N_PAD // R


def _mean(p, cnt):
    summed = jnp.concatenate([p[0], p[1]], axis=1)
    c = jnp.maximum(cnt[0], 1.0)
    return summed / c[:, None]


def _matT(a, w):
    return lax.dot_general(a, w, (((1,), (1,)), ((), ())),
                           preferred_element_type=jnp.float32)


def _l2relu(o):
    nrm = jnp.sqrt(jnp.sum(o * o, axis=1, keepdims=True))
    return jnp.maximum(o / jnp.maximum(nrm, 1e-12), 0.0)


def _halves(o):
    d = o.shape[1] // 2
    return jnp.stack([o[:, :d], o[:, d:]])


def _dense1_body(p_ref, cnt_ref, w1_ref, b1_ref, w2_ref, out_ref):
    mean = _mean(p_ref[...], cnt_ref[...])
    h = _l2relu(_matT(mean, w1_ref[...]) + b1_ref[...])
    out_ref[...] = _halves(_matT(h, w2_ref[...]))


def _dense2_body(p_ref, cnt_ref, b2_ref, out_ref):
    mean = _mean(p_ref[...], cnt_ref[...])
    out_ref[...] = _halves(_l2relu(mean + b2_ref[...]))


def _dense3_body(p_ref, cnt_ref, w3_ref, b3_ref, out_ref):
    mean = _mean(p_ref[...], cnt_ref[...])
    out_ref[...] = _halves(_l2relu(_matT(mean, w3_ref[...]) + b3_ref[...]))


def _dense4_body(p_ref, cnt_ref, w4_ref, b4_ref, out_ref):
    mean = _mean(p_ref[...], cnt_ref[...])
    o = _matT(mean, w4_ref[...]) + b4_ref[...]
    m = jnp.max(o, axis=1, keepdims=True)
    e = jnp.exp(o - m)
    out_ref[...] = e / jnp.sum(e, axis=1, keepdims=True)


def _run_dense(body, p, cnt, weights, d_out, split_out):
    dh_in = p.shape[-1]
    in_specs = ([pl.BlockSpec((NC, R, dh_in), lambda i: (0, i, 0)),
                 pl.BlockSpec((NC, R), lambda i: (0, i))]
                + [pl.BlockSpec(w.shape, lambda i, nd=len(w.shape): (0,) * nd)
                   for w in weights])
    if split_out:
        out_spec = pl.BlockSpec((NC, R, d_out // 2), lambda i: (0, i, 0))
        out_shape = jax.ShapeDtypeStruct((NC, N_PAD, d_out // 2), jnp.float32)
    else:
        out_spec = pl.BlockSpec((R, d_out), lambda i: (i, 0))
        out_shape = jax.ShapeDtypeStruct((N_PAD, d_out), jnp.float32)
    return pl.pallas_call(
        body,
        grid=(GRID,),
        in_specs=in_specs,
        out_specs=out_spec,
        out_shape=out_shape,
    )(p, cnt, *weights)


def kernel(x, edge_index, W1, b1, W2, b2, W3, b3, W4, b4):
    src = edge_index[0]
    dst = edge_index[1]
    e = src.shape[0]
    cpw = -(-(-(-e // K) // NS) // NB) * NB
    e_pad = cpw * NS * K
    pad = e_pad - e
    pad_src = jnp.arange(pad, dtype=jnp.int32) % N_NODES
    pad_dst = N_NODES + jnp.arange(pad, dtype=jnp.int32) % (N_PAD - N_NODES)
    src2 = jnp.concatenate([src, pad_src]).reshape(-1, K)
    dst2 = jnp.concatenate([dst, pad_dst]).reshape(-1, K)

    b1r = b1.reshape(1, -1)
    b2r = b2.reshape(1, -1)
    b3r = b3.reshape(1, -1)
    b4r = b4.reshape(1, -1)

    x01 = (x[:, :64], x[:, 64:])
    p0, cnt = _sc_aggregate(x01, src2, dst2, with_counts=True)
    p2in = _run_dense(_dense1_body, p0, cnt, [W1, b1r, W2], 64, True)
    q2 = _sc_aggregate((p2in[0], p2in[1]), src2, dst2)
    h2 = _run_dense(_dense2_body, q2, cnt, [b2r], 64, True)
    q3 = _sc_aggregate((h2[0], h2[1]), src2, dst2)
    h3 = _run_dense(_dense3_body, q3, cnt, [W3, b3r], 128, True)
    q4 = _sc_aggregate((h3[0], h3[1]), src2, dst2)
    out = _run_dense(_dense4_body, q4, cnt, [W4, b4r], 128, False)
    return out[:N_NODES]
